# baseline (speedup 1.0000x reference)
import functools

import jax
import jax.numpy as jnp
from jax import lax
from jax.experimental import pallas as pl
from jax.experimental.pallas import tpu as pltpu
from jax.experimental.pallas import tpu_sc as plsc

f32 = jnp.float32
bf16 = jnp.bfloat16

_N = 10000
_E = 160000
_D = 256
_NP = 10240
_W = 128
_WA = 160
_CH = 80
_NC, _NS = 2, 16
_EPT = _E // _NS
_NG = _EPT // _CH
_RPT = _NP // _NS


def _sc_aggregate(x01, ei):
  mesh = plsc.VectorSubcoreMesh(core_axis_name="c", subcore_axis_name="s")

  @functools.partial(
      pl.kernel,
      out_type=jax.ShapeDtypeStruct((_NC, _NP, _WA), bf16),
      mesh=mesh,
      scratch_types=[
          pltpu.VMEM((4, _CH), jnp.int32),
          pltpu.VMEM((4, _CH), jnp.int32),
          pltpu.VMEM((3, _CH, _WA), bf16),
          pltpu.VMEM_SHARED((_NP, _WA), bf16),
          pltpu.SemaphoreType.DMA((4,)),
          pltpu.SemaphoreType.DMA((3,)),
          pltpu.SemaphoreType.DMA((3,)),
      ],
      compiler_params=pltpu.CompilerParams(use_tc_tiling_on_sc=False),
  )
  def body(x01_hbm, ei_hbm, out_hbm, sidx, didx, rows3, acc,
           sem_i, sem_r, sem_w):
    cid = lax.axis_index("c")
    sid = lax.axis_index("s")

    def idx_start(g):
      pltpu.async_copy(ei_hbm.at[0, sid, g], sidx.at[g % 4], sem_i.at[g % 4])
      pltpu.async_copy(ei_hbm.at[1, sid, g], didx.at[g % 4], sem_i.at[g % 4])

    def idx_wait(g):
      pltpu.make_async_copy(ei_hbm.at[0, sid, g], sidx.at[g % 4],
                            sem_i.at[g % 4]).wait()
      pltpu.make_async_copy(ei_hbm.at[1, sid, g], didx.at[g % 4],
                            sem_i.at[g % 4]).wait()
      for k in range(_CH // 16):
        sl = pl.ds(k * 16, 16)
        sidx[g % 4, sl] = sidx[g % 4, sl] * 2 + cid

    def gather_start(g):
      pltpu.async_copy(x01_hbm.at[sidx.at[g % 4]], rows3.at[g % 3],
                       sem_r.at[g % 3])

    def gather_wait(g):
      pltpu.make_async_copy(x01_hbm.at[sidx.at[g % 4]], rows3.at[g % 3],
                            sem_r.at[g % 3]).wait()

    def scatter_start(g):
      pltpu.async_copy(rows3.at[g % 3], acc.at[didx.at[g % 4]],
                       sem_w.at[g % 3], add=True)

    def scatter_wait(g):
      pltpu.make_async_copy(rows3.at[g % 3], acc.at[didx.at[g % 4]],
                            sem_w.at[g % 3]).wait()

    idx_start(0)
    idx_start(1)

    zero32 = jnp.zeros((32,), bf16)
    rows0 = rows3.at[0]

    def zrows(k, c):
      rows0[k // (_WA // 32), pl.ds((k % (_WA // 32)) * 32, 32)] = zero32
      return c

    lax.fori_loop(0, _CH * (_WA // 32), zrows, 0)

    def zacc(k, c):
      pltpu.sync_copy(rows0, acc.at[pl.ds(sid * _RPT + k * _CH, _CH)])
      return c

    lax.fori_loop(0, _RPT // _CH, zacc, 0)
    plsc.subcore_barrier()

    idx_wait(0)
    gather_start(0)

    def step(g, c):
      gather_wait(g)

      @pl.when(g >= 2)
      def _():
        scatter_wait(g - 2)

      @pl.when(g + 2 < _NG)
      def _():
        idx_start(g + 2)

      @pl.when(g + 1 < _NG)
      def _():
        idx_wait(g + 1)
        gather_start(g + 1)

      scatter_start(g)
      return c

    lax.fori_loop(0, _NG, step, 0)
    scatter_wait(_NG - 2)
    scatter_wait(_NG - 1)
    plsc.subcore_barrier()

    pltpu.sync_copy(acc.at[pl.ds(sid * _RPT, _RPT)],
                    out_hbm.at[cid, pl.ds(sid * _RPT, _RPT)])

  return body(x01, ei)


def _tc_self(x, W_r, b_l):
  blk = 1000
  grid = (_N // blk,)

  def body(x_ref, wr_ref, b_ref, o_ref):
    xb = x_ref[...]
    o_ref[...] = (jnp.dot(xb, wr_ref[...], preferred_element_type=f32)
                  + xb + b_ref[...])

  return pl.pallas_call(
      body,
      grid=grid,
      in_specs=[
          pl.BlockSpec((blk, _D), lambda g: (g, 0)),
          pl.BlockSpec((_D, _D), lambda g: (0, 0)),
          pl.BlockSpec((1, _D), lambda g: (0, 0)),
      ],
      out_specs=pl.BlockSpec((blk, _D), lambda g: (g, 0)),
      out_shape=jax.ShapeDtypeStruct((_N, _D), f32),
  )(x, W_r, b_l.reshape(1, _D))


def _tc_dense(h, out01, W_l):
  blk = 1000
  grid = (_N // blk,)

  def body(a0_ref, a1_ref, h_ref, wl_ref, o_ref):
    a0b = a0_ref[0]
    a1b = a1_ref[0]
    cnt = a0b[:, 128:129].astype(f32)
    inv = 1.0 / jnp.maximum(cnt, 1.0)
    acc = jnp.dot(a0b[:, :_W] * inv, wl_ref[0:128, :],
                  preferred_element_type=f32)
    acc = acc + jnp.dot(a1b[:, :_W] * inv, wl_ref[128:256, :],
                        preferred_element_type=f32)
    o_ref[...] = acc + h_ref[...]

  return pl.pallas_call(
      body,
      grid=grid,
      in_specs=[
          pl.BlockSpec((1, blk, _WA), lambda g: (0, g, 0)),
          pl.BlockSpec((1, blk, _WA), lambda g: (1, g, 0)),
          pl.BlockSpec((blk, _D), lambda g: (g, 0)),
          pl.BlockSpec((_D, _D), lambda g: (0, 0)),
      ],
      out_specs=pl.BlockSpec((blk, _D), lambda g: (g, 0)),
      out_shape=jax.ShapeDtypeStruct((_N, _D), f32),
  )(out01, out01, h, W_l)


def kernel(x, edge_index, W_l, b_l, W_r):
  x01 = jnp.concatenate(
      [x.astype(bf16).reshape(2 * _N, _W),
       jnp.ones((2 * _N, _WA - _W), bf16)], axis=1)
  ei = edge_index.reshape(2, _NS, _NG, _CH)
  out01 = _sc_aggregate(x01, ei)
  h = _tc_self(x, W_r, b_l)
  return _tc_dense(h, out01, W_l)


# speedup vs baseline: 1.3411x; 1.3411x over previous
import functools

import jax
import jax.numpy as jnp
from jax import lax
from jax.experimental import pallas as pl
from jax.experimental.pallas import tpu as pltpu
from jax.experimental.pallas import tpu_sc as plsc

f32 = jnp.float32
bf16 = jnp.bfloat16

_N = 10000
_E = 160000
_D = 256
_NP = 10240
_W = 128
_CH = 80
_NC, _NS = 2, 16
_EPT = _E // _NS
_NG = _EPT // _CH
_RPT = _NP // _NS


def _sc_aggregate(x01, ei):
  mesh = plsc.VectorSubcoreMesh(core_axis_name="c", subcore_axis_name="s")

  @functools.partial(
      pl.kernel,
      out_type=[jax.ShapeDtypeStruct((_NC, _NP, _W), bf16),
                jax.ShapeDtypeStruct((_NC, _NP, 16), f32)],
      mesh=mesh,
      scratch_types=[
          pltpu.VMEM((4, _CH), jnp.int32),
          pltpu.VMEM((4, _CH), jnp.int32),
          pltpu.VMEM((3, _CH, _W), bf16),
          pltpu.VMEM((_CH, 16), f32),
          pltpu.VMEM_SHARED((_NP, _W), bf16),
          pltpu.VMEM_SHARED((_NP, 16), f32),
          pltpu.SemaphoreType.DMA((4,)),
          pltpu.SemaphoreType.DMA((3,)),
          pltpu.SemaphoreType.DMA((3,)),
      ],
      compiler_params=pltpu.CompilerParams(use_tc_tiling_on_sc=False),
  )
  def body(x01_hbm, ei_hbm, out_hbm, outc_hbm, sidx, didx, rows3,
           ones, acc, cnt, sem_i, sem_r, sem_w):
    cid = lax.axis_index("c")
    sid = lax.axis_index("s")

    def idx_start(g):
      pltpu.async_copy(ei_hbm.at[0, sid, g], sidx.at[g % 4], sem_i.at[g % 4])
      pltpu.async_copy(ei_hbm.at[1, sid, g], didx.at[g % 4], sem_i.at[g % 4])

    def idx_wait(g):
      pltpu.make_async_copy(ei_hbm.at[0, sid, g], sidx.at[g % 4],
                            sem_i.at[g % 4]).wait()
      pltpu.make_async_copy(ei_hbm.at[1, sid, g], didx.at[g % 4],
                            sem_i.at[g % 4]).wait()
      for k in range(_CH // 16):
        sl = pl.ds(k * 16, 16)
        sidx[g % 4, sl] = sidx[g % 4, sl] * 2 + cid

    def gather_start(g):
      pltpu.async_copy(x01_hbm.at[sidx.at[g % 4]], rows3.at[g % 3],
                       sem_r.at[g % 3])

    def gather_wait(g):
      pltpu.make_async_copy(x01_hbm.at[sidx.at[g % 4]], rows3.at[g % 3],
                            sem_r.at[g % 3]).wait()

    def scatter_start(g):
      pltpu.async_copy(rows3.at[g % 3], acc.at[didx.at[g % 4]],
                       sem_w.at[g % 3], add=True)

      @pl.when(g % 2 == cid)
      def _():
        pltpu.async_copy(ones, cnt.at[didx.at[g % 4]], sem_w.at[g % 3],
                         add=True)

    def scatter_wait(g):
      pltpu.make_async_copy(rows3.at[g % 3], acc.at[didx.at[g % 4]],
                            sem_w.at[g % 3]).wait()

      @pl.when(g % 2 == cid)
      def _():
        pltpu.make_async_copy(ones, cnt.at[didx.at[g % 4]],
                              sem_w.at[g % 3]).wait()

    idx_start(0)
    idx_start(1)

    zero = jnp.zeros((16,), f32)
    zero32 = jnp.zeros((32,), bf16)
    rows0 = rows3.at[0]

    def zrows(k, c):
      rows0[k // (_W // 32), pl.ds((k % (_W // 32)) * 32, 32)] = zero32
      return c

    lax.fori_loop(0, _CH * (_W // 32), zrows, 0)

    def zones(k, c):
      ones[k, pl.ds(0, 16)] = zero
      return c

    lax.fori_loop(0, _CH, zones, 0)

    def zacc(k, c):
      pltpu.sync_copy(rows0, acc.at[pl.ds(sid * _RPT + k * _CH, _CH)])
      return c

    lax.fori_loop(0, _RPT // _CH, zacc, 0)

    def zcnt(k, c):
      pltpu.sync_copy(ones, cnt.at[pl.ds(sid * _RPT + k * _CH, _CH)])
      return c

    lax.fori_loop(0, _RPT // _CH, zcnt, 0)

    one = jnp.ones((16,), f32)

    def fones(k, c):
      ones[k, pl.ds(0, 16)] = one
      return c

    lax.fori_loop(0, _CH, fones, 0)
    plsc.subcore_barrier()

    idx_wait(0)
    gather_start(0)

    def step(g, c):
      gather_wait(g)

      @pl.when(g >= 2)
      def _():
        scatter_wait(g - 2)

      @pl.when(g + 2 < _NG)
      def _():
        idx_start(g + 2)

      @pl.when(g + 1 < _NG)
      def _():
        idx_wait(g + 1)
        gather_start(g + 1)

      scatter_start(g)
      return c

    lax.fori_loop(0, _NG, step, 0)
    scatter_wait(_NG - 2)
    scatter_wait(_NG - 1)
    plsc.subcore_barrier()

    pltpu.sync_copy(acc.at[pl.ds(sid * _RPT, _RPT)],
                    out_hbm.at[cid, pl.ds(sid * _RPT, _RPT)])
    pltpu.sync_copy(cnt.at[pl.ds(sid * _RPT, _RPT)],
                    outc_hbm.at[cid, pl.ds(sid * _RPT, _RPT)])

  return body(x01, ei)


def _tc_self(x, W_r, b_l):
  blk = 1000
  grid = (_N // blk,)

  def body(x_ref, wr_ref, b_ref, o_ref):
    xb = x_ref[...]
    o_ref[...] = (jnp.dot(xb, wr_ref[...], preferred_element_type=f32)
                  + xb + b_ref[...])

  return pl.pallas_call(
      body,
      grid=grid,
      in_specs=[
          pl.BlockSpec((blk, _D), lambda g: (g, 0)),
          pl.BlockSpec((_D, _D), lambda g: (0, 0)),
          pl.BlockSpec((1, _D), lambda g: (0, 0)),
      ],
      out_specs=pl.BlockSpec((blk, _D), lambda g: (g, 0)),
      out_shape=jax.ShapeDtypeStruct((_N, _D), f32),
  )(x, W_r, b_l.reshape(1, _D))


def _tc_dense(h, out01, outc, W_l):
  blk = 1000
  grid = (_N // blk,)

  def body(a0_ref, a1_ref, c0_ref, c1_ref, h_ref, wl_ref, o_ref):
    a0b = a0_ref[0]
    a1b = a1_ref[0]
    cnt = c0_ref[0][:, 0:1] + c1_ref[0][:, 0:1]
    inv = 1.0 / jnp.maximum(cnt, 1.0)
    acc = jnp.dot(a0b * inv, wl_ref[0:128, :], preferred_element_type=f32)
    acc = acc + jnp.dot(a1b * inv, wl_ref[128:256, :],
                        preferred_element_type=f32)
    o_ref[...] = acc + h_ref[...]

  return pl.pallas_call(
      body,
      grid=grid,
      in_specs=[
          pl.BlockSpec((1, blk, _W), lambda g: (0, g, 0)),
          pl.BlockSpec((1, blk, _W), lambda g: (1, g, 0)),
          pl.BlockSpec((1, blk, 16), lambda g: (0, g, 0)),
          pl.BlockSpec((1, blk, 16), lambda g: (1, g, 0)),
          pl.BlockSpec((blk, _D), lambda g: (g, 0)),
          pl.BlockSpec((_D, _D), lambda g: (0, 0)),
      ],
      out_specs=pl.BlockSpec((blk, _D), lambda g: (g, 0)),
      out_shape=jax.ShapeDtypeStruct((_N, _D), f32),
  )(out01, out01, outc, outc, h, W_l)


def kernel(x, edge_index, W_l, b_l, W_r):
  x01 = x.astype(bf16).reshape(2 * _N, _W)
  ei = edge_index.reshape(2, _NS, _NG, _CH)
  out01, outc = _sc_aggregate(x01, ei)
  h = _tc_self(x, W_r, b_l)
  return _tc_dense(h, out01, outc, W_l)


# speedup vs baseline: 1.4880x; 1.1095x over previous
import functools

import jax
import jax.numpy as jnp
from jax import lax
from jax.experimental import pallas as pl
from jax.experimental.pallas import tpu as pltpu
from jax.experimental.pallas import tpu_sc as plsc

f32 = jnp.float32
bf16 = jnp.bfloat16

_N = 10000
_E = 160000
_D = 256
_NP = 10240
_CH = 80
_NC, _NS = 2, 16
_EPT = _E // _NS
_NG = _EPT // _CH
_RPT = _NP // _NS


def _sc_aggregate(xb, ei):
  mesh = plsc.VectorSubcoreMesh(core_axis_name="c", subcore_axis_name="s")

  @functools.partial(
      pl.kernel,
      out_type=[jax.ShapeDtypeStruct((_NC, _NP, _D), bf16),
                jax.ShapeDtypeStruct((_NC, _NP, 16), f32)],
      mesh=mesh,
      scratch_types=[
          pltpu.VMEM((4, _CH), jnp.int32),
          pltpu.VMEM((4, _CH), jnp.int32),
          pltpu.VMEM((3, _CH, _D), bf16),
          pltpu.VMEM((_CH, 16), f32),
          pltpu.VMEM_SHARED((_NP, _D), bf16),
          pltpu.VMEM_SHARED((_NP, 16), f32),
          pltpu.SemaphoreType.DMA((4,)),
          pltpu.SemaphoreType.DMA((3,)),
          pltpu.SemaphoreType.DMA((3,)),
      ],
      compiler_params=pltpu.CompilerParams(use_tc_tiling_on_sc=False),
  )
  def body(x_hbm, ei_hbm, out_hbm, outc_hbm, sidx, didx, rows3, ones,
           acc, cnt, sem_i, sem_r, sem_w):
    cid = lax.axis_index("c")
    sid = lax.axis_index("s")
    nb = (_NG + 1) // 2 - cid

    def idx_start(gg):
      g = 2 * gg + cid
      pltpu.async_copy(ei_hbm.at[0, sid, g], sidx.at[gg % 4],
                       sem_i.at[gg % 4])
      pltpu.async_copy(ei_hbm.at[1, sid, g], didx.at[gg % 4],
                       sem_i.at[gg % 4])

    def idx_wait(gg):
      g = 2 * gg + cid
      pltpu.make_async_copy(ei_hbm.at[0, sid, g], sidx.at[gg % 4],
                            sem_i.at[gg % 4]).wait()
      pltpu.make_async_copy(ei_hbm.at[1, sid, g], didx.at[gg % 4],
                            sem_i.at[gg % 4]).wait()

    def gather_start(gg):
      pltpu.async_copy(x_hbm.at[sidx.at[gg % 4]], rows3.at[gg % 3],
                       sem_r.at[gg % 3])

    def gather_wait(gg):
      pltpu.make_async_copy(x_hbm.at[sidx.at[gg % 4]], rows3.at[gg % 3],
                            sem_r.at[gg % 3]).wait()

    def scatter_start(gg):
      pltpu.async_copy(rows3.at[gg % 3], acc.at[didx.at[gg % 4]],
                       sem_w.at[gg % 3], add=True)
      pltpu.async_copy(ones, cnt.at[didx.at[gg % 4]], sem_w.at[gg % 3],
                       add=True)

    def scatter_wait(gg):
      pltpu.make_async_copy(rows3.at[gg % 3], acc.at[didx.at[gg % 4]],
                            sem_w.at[gg % 3]).wait()
      pltpu.make_async_copy(ones, cnt.at[didx.at[gg % 4]],
                            sem_w.at[gg % 3]).wait()

    idx_start(0)
    idx_start(1)

    zero = jnp.zeros((16,), f32)
    zero32 = jnp.zeros((32,), bf16)
    rows0 = rows3.at[0]

    def zrows(k, c):
      rows0[k // (_D // 32), pl.ds((k % (_D // 32)) * 32, 32)] = zero32
      return c

    lax.fori_loop(0, _CH * (_D // 32), zrows, 0)

    def zones(k, c):
      ones[k, pl.ds(0, 16)] = zero
      return c

    lax.fori_loop(0, _CH, zones, 0)

    def zacc(k, c):
      pltpu.sync_copy(rows0, acc.at[pl.ds(sid * _RPT + k * _CH, _CH)])
      return c

    lax.fori_loop(0, _RPT // _CH, zacc, 0)

    def zcnt(k, c):
      pltpu.sync_copy(ones, cnt.at[pl.ds(sid * _RPT + k * _CH, _CH)])
      return c

    lax.fori_loop(0, _RPT // _CH, zcnt, 0)

    one = jnp.ones((16,), f32)

    def fones(k, c):
      ones[k, pl.ds(0, 16)] = one
      return c

    lax.fori_loop(0, _CH, fones, 0)
    plsc.subcore_barrier()

    idx_wait(0)
    gather_start(0)

    def step(gg, c):
      gather_wait(gg)

      @pl.when(gg >= 2)
      def _():
        scatter_wait(gg - 2)

      @pl.when(gg + 2 < nb)
      def _():
        idx_start(gg + 2)

      @pl.when(gg + 1 < nb)
      def _():
        idx_wait(gg + 1)
        gather_start(gg + 1)

      scatter_start(gg)
      return c

    lax.fori_loop(0, nb, step, 0)
    scatter_wait(nb - 2)
    scatter_wait(nb - 1)
    plsc.subcore_barrier()

    pltpu.sync_copy(acc.at[pl.ds(sid * _RPT, _RPT)],
                    out_hbm.at[cid, pl.ds(sid * _RPT, _RPT)])
    pltpu.sync_copy(cnt.at[pl.ds(sid * _RPT, _RPT)],
                    outc_hbm.at[cid, pl.ds(sid * _RPT, _RPT)])

  return body(xb, ei)


def _tc_self(x, W_r, b_l):
  blk = 1000
  grid = (_N // blk,)

  def body(x_ref, wr_ref, b_ref, o_ref):
    xb = x_ref[...]
    o_ref[...] = (jnp.dot(xb, wr_ref[...], preferred_element_type=f32)
                  + xb + b_ref[...])

  return pl.pallas_call(
      body,
      grid=grid,
      in_specs=[
          pl.BlockSpec((blk, _D), lambda g: (g, 0)),
          pl.BlockSpec((_D, _D), lambda g: (0, 0)),
          pl.BlockSpec((1, _D), lambda g: (0, 0)),
      ],
      out_specs=pl.BlockSpec((blk, _D), lambda g: (g, 0)),
      out_shape=jax.ShapeDtypeStruct((_N, _D), f32),
  )(x, W_r, b_l.reshape(1, _D))


def _tc_dense(h, out01, outc, W_l):
  blk = 1000
  grid = (_N // blk,)

  def body(a0_ref, a1_ref, c0_ref, c1_ref, h_ref, wl_ref, o_ref):
    agg = a0_ref[0].astype(f32) + a1_ref[0].astype(f32)
    cnt = c0_ref[0][:, 0:1] + c1_ref[0][:, 0:1]
    inv = 1.0 / jnp.maximum(cnt, 1.0)
    acc = jnp.dot(agg * inv, wl_ref[...], preferred_element_type=f32)
    o_ref[...] = acc + h_ref[...]

  return pl.pallas_call(
      body,
      grid=grid,
      in_specs=[
          pl.BlockSpec((1, blk, _D), lambda g: (0, g, 0)),
          pl.BlockSpec((1, blk, _D), lambda g: (1, g, 0)),
          pl.BlockSpec((1, blk, 16), lambda g: (0, g, 0)),
          pl.BlockSpec((1, blk, 16), lambda g: (1, g, 0)),
          pl.BlockSpec((blk, _D), lambda g: (g, 0)),
          pl.BlockSpec((_D, _D), lambda g: (0, 0)),
      ],
      out_specs=pl.BlockSpec((blk, _D), lambda g: (g, 0)),
      out_shape=jax.ShapeDtypeStruct((_N, _D), f32),
  )(out01, out01, outc, outc, h, W_l)


def kernel(x, edge_index, W_l, b_l, W_r):
  xb = x.astype(bf16)
  ei = edge_index.reshape(2, _NS, _NG, _CH)
  out01, outc = _sc_aggregate(xb, ei)
  h = _tc_self(x, W_r, b_l)
  return _tc_dense(h, out01, outc, W_l)
